# baseline (speedup 1.0000x reference)
import jax
import jax.numpy as jnp
from jax import lax
from jax.experimental import pallas as pl
from jax.experimental.pallas import tpu as pltpu
from jax.experimental.pallas import tpu_sc as plsc

N = 10000
D_IN = 128
D_HID = 128
D_OUT = 64
E = 320000

NP = 10240
SP = NP // 16
NW = 32
CHUNK = 128
CPW = 80
HCH = CPW // 2
EPW = CHUNK * CPW
EP = EPW * NW

_MESH = plsc.VectorSubcoreMesh(core_axis_name="c", subcore_axis_name="s")


def _deg_body(dstA, dstB, out, idst, ones_v, zer_v, ssem, acc):
    c = lax.axis_index("c")
    s = lax.axis_index("s")
    wid = s * 2 + c

    def fill_ones(j, carry):
        ones_v[pl.ds(j * 16, 16)] = jnp.ones((16,), jnp.float32)
        return carry

    lax.fori_loop(0, CHUNK // 16, fill_ones, 0)

    def fill_zer(j, carry):
        zer_v[pl.ds(j * 16, 16)] = jnp.zeros((16,), jnp.float32)
        return carry

    lax.fori_loop(0, SP // 16, fill_zer, 0)
    pltpu.sync_copy(zer_v, acc.at[pl.ds(s * SP, SP)])
    plsc.subcore_barrier()

    for dstH in (dstA, dstB):
        pltpu.sync_copy(dstH.at[wid], idst)

        def fire(i, carry):
            pltpu.async_copy(ones_v, acc.at[idst.at[i]], ssem, add=True)
            return carry

        lax.fori_loop(0, HCH, fire, 0)

        def drain(i, carry):
            pltpu.make_async_copy(ones_v, acc.at[idst.at[0]], ssem).wait()
            return carry

        lax.fori_loop(0, HCH, drain, 0)
    plsc.subcore_barrier()
    pltpu.sync_copy(acc.at[pl.ds(s * SP, SP)], out.at[c, pl.ds(s * SP, SP)])


_deg_call = pl.kernel(
    _deg_body,
    out_type=jax.ShapeDtypeStruct((2, NP), jnp.float32),
    mesh=_MESH,
    scratch_types=[
        pltpu.VMEM((HCH, CHUNK), jnp.int32),
        pltpu.VMEM((CHUNK,), jnp.float32),
        pltpu.VMEM((SP,), jnp.float32),
        pltpu.SemaphoreType.DMA,
        pltpu.VMEM_SHARED((NP,), jnp.float32),
    ],
)


def _agg_body(hs, srcA, srcB, dstA, dstB, zinit, out, isrc, idst, rows0, rows1,
              gsem0, gsem1, ssem0, ssem1, acc):
    c = lax.axis_index("c")
    s = lax.axis_index("s")
    wid = s * 2 + c
    stripe = pl.ds(s * SP, SP)

    pltpu.sync_copy(srcA.at[wid], isrc)
    pltpu.sync_copy(dstA.at[wid], idst)
    pltpu.async_copy(hs.at[isrc.at[0]], rows0, gsem0)

    @pl.when(c == 0)
    def _():
        pltpu.sync_copy(hs.at[stripe], acc.at[stripe])

    @pl.when(c == 1)
    def _():
        pltpu.sync_copy(zinit.at[stripe], acc.at[stripe])

    plsc.subcore_barrier()

    for h, (srcH, dstH) in enumerate(((srcA, dstA), (srcB, dstB))):
        if h == 1:
            pltpu.sync_copy(srcH.at[wid], isrc)
            pltpu.sync_copy(dstH.at[wid], idst)
            pltpu.async_copy(hs.at[isrc.at[0]], rows0, gsem0)

        def pair(p, carry):
            k = 2 * p
            pltpu.make_async_copy(hs.at[isrc.at[k]], rows0, gsem0).wait()
            pltpu.async_copy(rows0, acc.at[idst.at[k]], ssem0, add=True)

            @pl.when(p > 0)
            def _():
                pltpu.make_async_copy(rows1, acc.at[idst.at[k - 1]],
                                      ssem1).wait()

            pltpu.async_copy(hs.at[isrc.at[k + 1]], rows1, gsem1)
            pltpu.make_async_copy(hs.at[isrc.at[k + 1]], rows1, gsem1).wait()
            pltpu.async_copy(rows1, acc.at[idst.at[k + 1]], ssem1, add=True)
            pltpu.make_async_copy(rows0, acc.at[idst.at[k]], ssem0).wait()

            @pl.when(p < HCH // 2 - 1)
            def _():
                pltpu.async_copy(hs.at[isrc.at[k + 2]], rows0, gsem0)

            return carry

        lax.fori_loop(0, HCH // 2, pair, 0)
        pltpu.make_async_copy(rows1, acc.at[idst.at[HCH - 1]], ssem1).wait()
    plsc.subcore_barrier()
    pltpu.sync_copy(acc.at[stripe], out.at[c, stripe])


_agg128 = pl.kernel(
    _agg_body,
    out_type=jax.ShapeDtypeStruct((2, NP, D_HID), jnp.float32),
    mesh=_MESH,
    scratch_types=[
        pltpu.VMEM((HCH, CHUNK), jnp.int32),
        pltpu.VMEM((HCH, CHUNK), jnp.int32),
        pltpu.VMEM((CHUNK, D_HID), jnp.float32),
        pltpu.VMEM((CHUNK, D_HID), jnp.float32),
        pltpu.SemaphoreType.DMA,
        pltpu.SemaphoreType.DMA,
        pltpu.SemaphoreType.DMA,
        pltpu.SemaphoreType.DMA,
        pltpu.VMEM_SHARED((NP, D_HID), jnp.float32),
    ],
)

R = 1280
GRID = NP // R


def _dinv(p0_ref, p1_ref):
    return lax.rsqrt(p0_ref[...] + p1_ref[...] + 1.0)


def _tc_first_body(x_ref, w_ref, p0_ref, p1_ref, o_ref):
    y = jnp.dot(x_ref[...], w_ref[...], preferred_element_type=jnp.float32)
    o_ref[...] = y * _dinv(p0_ref, p1_ref)


def _tc_mid_body(part_ref, p0_ref, p1_ref, b_ref, w_ref, o_ref):
    dinv = _dinv(p0_ref, p1_ref)
    t = (part_ref[0] + part_ref[1]) * dinv + b_ref[...]
    h = jnp.maximum(t, 0.0)
    o_ref[...] = jnp.dot(h, w_ref[...], preferred_element_type=jnp.float32) * dinv


def _tc_fin_body(part_ref, p0_ref, p1_ref, b_ref, o_ref):
    dinv = _dinv(p0_ref, p1_ref)
    t = part_ref[0, :, :D_OUT] + part_ref[1, :, :D_OUT]
    o_ref[...] = t * dinv + b_ref[...]


def _row_spec(d):
    return pl.BlockSpec((R, d), lambda i: (i, 0))


def _tc_first(x_p, w, p0r, p1r):
    d_in, d_out = w.shape
    return pl.pallas_call(
        _tc_first_body,
        grid=(GRID,),
        in_specs=[
            _row_spec(d_in),
            pl.BlockSpec((d_in, d_out), lambda i: (0, 0)),
            pl.BlockSpec((R, 1), lambda i: (i, 0)),
            pl.BlockSpec((R, 1), lambda i: (i, 0)),
        ],
        out_specs=_row_spec(d_out),
        out_shape=jax.ShapeDtypeStruct((NP, d_out), jnp.float32),
    )(x_p, w, p0r, p1r)


def _tc_mid(part, p0r, p1r, b, w):
    d_in, d_out = w.shape
    return pl.pallas_call(
        _tc_mid_body,
        grid=(GRID,),
        in_specs=[
            pl.BlockSpec((2, R, d_in), lambda i: (0, i, 0)),
            pl.BlockSpec((R, 1), lambda i: (i, 0)),
            pl.BlockSpec((R, 1), lambda i: (i, 0)),
            pl.BlockSpec((1, d_in), lambda i: (0, 0)),
            pl.BlockSpec((d_in, d_out), lambda i: (0, 0)),
        ],
        out_specs=_row_spec(d_out),
        out_shape=jax.ShapeDtypeStruct((NP, d_out), jnp.float32),
    )(part, p0r, p1r, b, w)


def _tc_fin(part, p0r, p1r, b):
    d = part.shape[-1]
    return pl.pallas_call(
        _tc_fin_body,
        grid=(GRID,),
        in_specs=[
            pl.BlockSpec((2, R, d), lambda i: (0, i, 0)),
            pl.BlockSpec((R, 1), lambda i: (i, 0)),
            pl.BlockSpec((R, 1), lambda i: (i, 0)),
            pl.BlockSpec((1, D_OUT), lambda i: (0, 0)),
        ],
        out_specs=_row_spec(D_OUT),
        out_shape=jax.ShapeDtypeStruct((NP, D_OUT), jnp.float32),
    )(part, p0r, p1r, b)


def kernel(x, edge_index, W1, b1, W2, b2, W3, b3):
    f32 = jnp.float32
    x_p = jnp.zeros((NP, D_IN), f32).at[:N].set(x)
    pad = EP - E
    pad_ar = jnp.arange(pad, dtype=jnp.int32)
    src4 = jnp.concatenate([edge_index[0], pad_ar % N]).reshape(NW, 2, HCH, CHUNK)
    dst4 = jnp.concatenate([edge_index[1], N + (pad_ar % 16)]).reshape(NW, 2, HCH, CHUNK)
    srcA, srcB = src4[:, 0], src4[:, 1]
    dstA, dstB = dst4[:, 0], dst4[:, 1]
    z128 = jnp.zeros((NP, D_HID), f32)
    W3p = jnp.zeros((D_HID, D_HID), f32).at[:, :D_OUT].set(W3)

    degp = _deg_call(dstA, dstB)
    p0r = degp[0][:, None]
    p1r = degp[1][:, None]

    hs1 = _tc_first(x_p, W1, p0r, p1r)
    part1 = _agg128(hs1, srcA, srcB, dstA, dstB, z128)
    hs2 = _tc_mid(part1, p0r, p1r, b1.reshape(1, -1), W2)
    part2 = _agg128(hs2, srcA, srcB, dstA, dstB, z128)
    hs3 = _tc_mid(part2, p0r, p1r, b2.reshape(1, -1), W3p)
    part3 = _agg128(hs3, srcA, srcB, dstA, dstB, z128)
    out = _tc_fin(part3, p0r, p1r, b3.reshape(1, -1))
    return out[:N]


# speedup vs baseline: 1.1718x; 1.1718x over previous
import jax
import jax.numpy as jnp
from jax import lax
from jax.experimental import pallas as pl
from jax.experimental.pallas import tpu as pltpu
from jax.experimental.pallas import tpu_sc as plsc

N = 10000
D_IN = 128
D_HID = 128
D_OUT = 64
E = 320000

NP = 10240
SP = NP // 16
NW = 32
CHUNK = 128
CPW = 80
HCH = CPW // 2
EPW = CHUNK * CPW
EP = EPW * NW

_MESH = plsc.VectorSubcoreMesh(core_axis_name="c", subcore_axis_name="s")


def _deg_body(dstA, dstB, out, idst, ones_v, zer_v, ssem, acc):
    c = lax.axis_index("c")
    s = lax.axis_index("s")
    wid = s * 2 + c

    def fill_ones(j, carry):
        ones_v[pl.ds(j * 16, 16)] = jnp.ones((16,), jnp.float32)
        return carry

    lax.fori_loop(0, CHUNK // 16, fill_ones, 0)

    def fill_zer(j, carry):
        zer_v[pl.ds(j * 16, 16)] = jnp.zeros((16,), jnp.float32)
        return carry

    lax.fori_loop(0, SP // 16, fill_zer, 0)
    pltpu.sync_copy(zer_v, acc.at[pl.ds(s * SP, SP)])
    plsc.subcore_barrier()

    for dstH in (dstA, dstB):
        pltpu.sync_copy(dstH.at[wid], idst)

        def fire(i, carry):
            pltpu.async_copy(ones_v, acc.at[idst.at[i]], ssem, add=True)
            return carry

        lax.fori_loop(0, HCH, fire, 0)

        def drain(i, carry):
            pltpu.make_async_copy(ones_v, acc.at[idst.at[0]], ssem).wait()
            return carry

        lax.fori_loop(0, HCH, drain, 0)
    plsc.subcore_barrier()
    pltpu.sync_copy(acc.at[pl.ds(s * SP, SP)], out.at[c, pl.ds(s * SP, SP)])


_deg_call = pl.kernel(
    _deg_body,
    out_type=jax.ShapeDtypeStruct((2, NP), jnp.float32),
    mesh=_MESH,
    scratch_types=[
        pltpu.VMEM((HCH, CHUNK), jnp.int32),
        pltpu.VMEM((CHUNK,), jnp.float32),
        pltpu.VMEM((SP,), jnp.float32),
        pltpu.SemaphoreType.DMA,
        pltpu.VMEM_SHARED((NP,), jnp.float32),
    ],
)


def _agg_body(hs, srcA, srcB, dstA, dstB, zinit, out, isrc, idst, rows0, rows1,
              gsem0, gsem1, ssem0, ssem1, acc):
    c = lax.axis_index("c")
    s = lax.axis_index("s")
    wid = s * 2 + c
    stripe = pl.ds(s * SP, SP)

    pltpu.sync_copy(srcA.at[wid], isrc)
    pltpu.sync_copy(dstA.at[wid], idst)
    pltpu.async_copy(hs.at[isrc.at[0]], rows0, gsem0)

    @pl.when(c == 0)
    def _():
        pltpu.sync_copy(hs.at[stripe], acc.at[stripe])

    @pl.when(c == 1)
    def _():
        pltpu.sync_copy(zinit.at[stripe], acc.at[stripe])

    plsc.subcore_barrier()

    for h, (srcH, dstH) in enumerate(((srcA, dstA), (srcB, dstB))):
        if h == 1:
            pltpu.sync_copy(srcH.at[wid], isrc)
            pltpu.sync_copy(dstH.at[wid], idst)
            pltpu.async_copy(hs.at[isrc.at[0]], rows0, gsem0)

        def pair(p, carry):
            k = 2 * p
            pltpu.async_copy(hs.at[isrc.at[k + 1]], rows1, gsem1)
            pltpu.make_async_copy(hs.at[isrc.at[k]], rows0, gsem0).wait()
            pltpu.sync_copy(rows0, acc.at[idst.at[k]], add=True)

            @pl.when(p < HCH // 2 - 1)
            def _():
                pltpu.async_copy(hs.at[isrc.at[k + 2]], rows0, gsem0)

            pltpu.make_async_copy(hs.at[isrc.at[k + 1]], rows1, gsem1).wait()
            pltpu.sync_copy(rows1, acc.at[idst.at[k + 1]], add=True)
            return carry

        lax.fori_loop(0, HCH // 2, pair, 0)
    plsc.subcore_barrier()
    pltpu.sync_copy(acc.at[stripe], out.at[c, stripe])


_agg128 = pl.kernel(
    _agg_body,
    out_type=jax.ShapeDtypeStruct((2, NP, D_HID), jnp.float32),
    mesh=_MESH,
    scratch_types=[
        pltpu.VMEM((HCH, CHUNK), jnp.int32),
        pltpu.VMEM((HCH, CHUNK), jnp.int32),
        pltpu.VMEM((CHUNK, D_HID), jnp.float32),
        pltpu.VMEM((CHUNK, D_HID), jnp.float32),
        pltpu.SemaphoreType.DMA,
        pltpu.SemaphoreType.DMA,
        pltpu.SemaphoreType.DMA,
        pltpu.SemaphoreType.DMA,
        pltpu.VMEM_SHARED((NP, D_HID), jnp.float32),
    ],
)

R = 1280
GRID = NP // R


def _tc_first_body(x_ref, w_ref, dg_ref, o_ref):
    y = jnp.dot(x_ref[...], w_ref[...], preferred_element_type=jnp.float32)
    o_ref[...] = y * lax.rsqrt(dg_ref[...])


def _tc_mid_body(part_ref, dg_ref, b_ref, w_ref, o_ref):
    dinv = lax.rsqrt(dg_ref[...])
    t = (part_ref[0] + part_ref[1]) * dinv + b_ref[...]
    h = jnp.maximum(t, 0.0)
    o_ref[...] = jnp.dot(h, w_ref[...], preferred_element_type=jnp.float32) * dinv


def _tc_fin_body(part_ref, dg_ref, b_ref, o_ref):
    dinv = lax.rsqrt(dg_ref[...])
    t = part_ref[0, :, :D_OUT] + part_ref[1, :, :D_OUT]
    o_ref[...] = t * dinv + b_ref[...]


def _row_spec(d):
    return pl.BlockSpec((R, d), lambda i: (i, 0))


def _tc_first(x_p, w, dgr):
    d_in, d_out = w.shape
    return pl.pallas_call(
        _tc_first_body,
        grid=(GRID,),
        in_specs=[
            _row_spec(d_in),
            pl.BlockSpec((d_in, d_out), lambda i: (0, 0)),
            pl.BlockSpec((R, 1), lambda i: (i, 0)),
        ],
        out_specs=_row_spec(d_out),
        out_shape=jax.ShapeDtypeStruct((NP, d_out), jnp.float32),
    )(x_p, w, dgr)


def _tc_mid(part, dgr, b, w):
    d_in, d_out = w.shape
    return pl.pallas_call(
        _tc_mid_body,
        grid=(GRID,),
        in_specs=[
            pl.BlockSpec((2, R, d_in), lambda i: (0, i, 0)),
            pl.BlockSpec((R, 1), lambda i: (i, 0)),
            pl.BlockSpec((1, d_in), lambda i: (0, 0)),
            pl.BlockSpec((d_in, d_out), lambda i: (0, 0)),
        ],
        out_specs=_row_spec(d_out),
        out_shape=jax.ShapeDtypeStruct((NP, d_out), jnp.float32),
    )(part, dgr, b, w)


RF = 1000


def _tc_fin(part, dgr, b):
    d = part.shape[-1]
    return pl.pallas_call(
        _tc_fin_body,
        grid=(N // RF,),
        in_specs=[
            pl.BlockSpec((2, RF, d), lambda i: (0, i, 0)),
            pl.BlockSpec((RF, 1), lambda i: (i, 0)),
            pl.BlockSpec((1, D_OUT), lambda i: (0, 0)),
        ],
        out_specs=pl.BlockSpec((RF, D_OUT), lambda i: (i, 0)),
        out_shape=jax.ShapeDtypeStruct((N, D_OUT), jnp.float32),
    )(part, dgr, b)


def kernel(x, edge_index, W1, b1, W2, b2, W3, b3):
    f32 = jnp.float32
    x_p = jnp.zeros((NP, D_IN), f32).at[:N].set(x)
    pad = EP - E
    pad_ar = jnp.arange(pad, dtype=jnp.int32)
    src4 = jnp.concatenate([edge_index[0], pad_ar % N]).reshape(NW, 2, HCH, CHUNK)
    dst4 = jnp.concatenate([edge_index[1], N + (pad_ar % 16)]).reshape(NW, 2, HCH, CHUNK)
    srcA, srcB = src4[:, 0], src4[:, 1]
    dstA, dstB = dst4[:, 0], dst4[:, 1]
    z128 = jnp.zeros((NP, D_HID), f32)
    W3p = jnp.zeros((D_HID, D_HID), f32).at[:, :D_OUT].set(W3)

    degp = _deg_call(dstA, dstB)
    dgr = (degp[0] + degp[1] + 1.0)[:, None]

    hs1 = _tc_first(x_p, W1, dgr)
    part1 = _agg128(hs1, srcA, srcB, dstA, dstB, z128)
    hs2 = _tc_mid(part1, dgr, b1.reshape(1, -1), W2)
    part2 = _agg128(hs2, srcA, srcB, dstA, dstB, z128)
    hs3 = _tc_mid(part2, dgr, b2.reshape(1, -1), W3p)
    part3 = _agg128(hs3, srcA, srcB, dstA, dstB, z128)
    return _tc_fin(part3, dgr, b3.reshape(1, -1))


# speedup vs baseline: 1.2455x; 1.0629x over previous
import jax
import jax.numpy as jnp
from jax import lax
from jax.experimental import pallas as pl
from jax.experimental.pallas import tpu as pltpu
from jax.experimental.pallas import tpu_sc as plsc

N = 10000
D_IN = 128
D_HID = 128
D_OUT = 64
E = 320000

NP = 10240
SP = NP // 16
NW = 32
CHUNK = 128
CPW = 80
HCH = CPW // 2
EPW = CHUNK * CPW
EP = EPW * NW

_MESH = plsc.VectorSubcoreMesh(core_axis_name="c", subcore_axis_name="s")


def _deg_body(dstA, dstB, out, idst, ones_v, zer_v, ssem, acc):
    c = lax.axis_index("c")
    s = lax.axis_index("s")
    wid = s * 2 + c

    def fill_ones(j, carry):
        ones_v[pl.ds(j * 16, 16)] = jnp.ones((16,), jnp.float32)
        return carry

    lax.fori_loop(0, CHUNK // 16, fill_ones, 0)

    def fill_zer(j, carry):
        zer_v[pl.ds(j * 16, 16)] = jnp.zeros((16,), jnp.float32)
        return carry

    lax.fori_loop(0, SP // 16, fill_zer, 0)
    pltpu.sync_copy(zer_v, acc.at[pl.ds(s * SP, SP)])
    plsc.subcore_barrier()

    for dstH in (dstA, dstB):
        pltpu.sync_copy(dstH.at[wid], idst)

        def fire(i, carry):
            pltpu.async_copy(ones_v, acc.at[idst.at[i]], ssem, add=True)
            return carry

        lax.fori_loop(0, HCH, fire, 0)

        def drain(i, carry):
            pltpu.make_async_copy(ones_v, acc.at[idst.at[0]], ssem).wait()
            return carry

        lax.fori_loop(0, HCH, drain, 0)
    plsc.subcore_barrier()
    pltpu.sync_copy(acc.at[pl.ds(s * SP, SP)], out.at[c, pl.ds(s * SP, SP)])


_deg_call = pl.kernel(
    _deg_body,
    out_type=jax.ShapeDtypeStruct((2, NP), jnp.float32),
    mesh=_MESH,
    scratch_types=[
        pltpu.VMEM((HCH, CHUNK), jnp.int32),
        pltpu.VMEM((CHUNK,), jnp.float32),
        pltpu.VMEM((SP,), jnp.float32),
        pltpu.SemaphoreType.DMA,
        pltpu.VMEM_SHARED((NP,), jnp.float32),
    ],
)


def _agg_body(hs, srcA, srcB, dstA, dstB, zinit, out, isrc, idst, rows0, rows1,
              gsem0, gsem1, ssem0, ssem1, acc):
    c = lax.axis_index("c")
    s = lax.axis_index("s")
    wid = s * 2 + c
    stripe = pl.ds(s * SP, SP)

    pltpu.sync_copy(srcA.at[wid], isrc)
    pltpu.sync_copy(dstA.at[wid], idst)
    pltpu.async_copy(hs.at[isrc.at[0]], rows0, gsem0)

    @pl.when(c == 0)
    def _():
        pltpu.sync_copy(hs.at[stripe], acc.at[stripe])

    @pl.when(c == 1)
    def _():
        pltpu.sync_copy(zinit.at[stripe], acc.at[stripe])

    plsc.subcore_barrier()

    for h, (srcH, dstH) in enumerate(((srcA, dstA), (srcB, dstB))):
        if h == 1:
            pltpu.sync_copy(srcH.at[wid], isrc)
            pltpu.sync_copy(dstH.at[wid], idst)
            pltpu.async_copy(hs.at[isrc.at[0]], rows0, gsem0)

        def pair(p, carry):
            k = 2 * p
            pltpu.async_copy(hs.at[isrc.at[k + 1]], rows1, gsem1)
            pltpu.make_async_copy(hs.at[isrc.at[k]], rows0, gsem0).wait()
            pltpu.sync_copy(rows0, acc.at[idst.at[k]], add=True)

            @pl.when(p < HCH // 2 - 1)
            def _():
                pltpu.async_copy(hs.at[isrc.at[k + 2]], rows0, gsem0)

            pltpu.make_async_copy(hs.at[isrc.at[k + 1]], rows1, gsem1).wait()
            pltpu.sync_copy(rows1, acc.at[idst.at[k + 1]], add=True)
            return carry

        lax.fori_loop(0, HCH // 2, pair, 0)
    plsc.subcore_barrier()
    pltpu.sync_copy(acc.at[stripe], out.at[c, stripe])


def _make_agg(d, **kw):
    return pl.kernel(
        _agg_body,
        out_type=jax.ShapeDtypeStruct((2, NP, d), jnp.float32),
        mesh=_MESH,
        scratch_types=[
            pltpu.VMEM((HCH, CHUNK), jnp.int32),
            pltpu.VMEM((HCH, CHUNK), jnp.int32),
            pltpu.VMEM((CHUNK, d), jnp.float32),
            pltpu.VMEM((CHUNK, d), jnp.float32),
            pltpu.SemaphoreType.DMA,
            pltpu.SemaphoreType.DMA,
            pltpu.SemaphoreType.DMA,
            pltpu.SemaphoreType.DMA,
            pltpu.VMEM_SHARED((NP, d), jnp.float32),
        ],
        **kw,
    )


_agg128 = _make_agg(D_HID)
_agg64 = _make_agg(
    D_OUT,
    compiler_params=pltpu.CompilerParams(use_tc_tiling_on_sc=False),
)

R = 1280
GRID = NP // R


def _tc_first_body(x_ref, w_ref, dg_ref, o_ref):
    y = jnp.dot(x_ref[...], w_ref[...], preferred_element_type=jnp.float32)
    o_ref[...] = y * lax.rsqrt(dg_ref[...])


def _tc_mid_body(part_ref, dg_ref, b_ref, w_ref, o_ref):
    dinv = lax.rsqrt(dg_ref[...])
    t = (part_ref[0] + part_ref[1]) * dinv + b_ref[...]
    h = jnp.maximum(t, 0.0)
    o_ref[...] = jnp.dot(h, w_ref[...], preferred_element_type=jnp.float32) * dinv


def _tc_fin_body(part_ref, dg_ref, b_ref, o_ref):
    dinv = lax.rsqrt(dg_ref[...])
    o_ref[...] = (part_ref[0] + part_ref[1]) * dinv + b_ref[...]


def _row_spec(d):
    return pl.BlockSpec((R, d), lambda i: (i, 0))


def _tc_first(x_p, w, dgr):
    d_in, d_out = w.shape
    return pl.pallas_call(
        _tc_first_body,
        grid=(GRID,),
        in_specs=[
            _row_spec(d_in),
            pl.BlockSpec((d_in, d_out), lambda i: (0, 0)),
            pl.BlockSpec((R, 1), lambda i: (i, 0)),
        ],
        out_specs=_row_spec(d_out),
        out_shape=jax.ShapeDtypeStruct((NP, d_out), jnp.float32),
    )(x_p, w, dgr)


def _tc_mid(part, dgr, b, w):
    d_in, d_out = w.shape
    return pl.pallas_call(
        _tc_mid_body,
        grid=(GRID,),
        in_specs=[
            pl.BlockSpec((2, R, d_in), lambda i: (0, i, 0)),
            pl.BlockSpec((R, 1), lambda i: (i, 0)),
            pl.BlockSpec((1, d_in), lambda i: (0, 0)),
            pl.BlockSpec((d_in, d_out), lambda i: (0, 0)),
        ],
        out_specs=_row_spec(d_out),
        out_shape=jax.ShapeDtypeStruct((NP, d_out), jnp.float32),
    )(part, dgr, b, w)


RF = 1000


def _tc_fin(part, dgr, b):
    d = part.shape[-1]
    return pl.pallas_call(
        _tc_fin_body,
        grid=(N // RF,),
        in_specs=[
            pl.BlockSpec((2, RF, d), lambda i: (0, i, 0)),
            pl.BlockSpec((RF, 1), lambda i: (i, 0)),
            pl.BlockSpec((1, D_OUT), lambda i: (0, 0)),
        ],
        out_specs=pl.BlockSpec((RF, D_OUT), lambda i: (i, 0)),
        out_shape=jax.ShapeDtypeStruct((N, D_OUT), jnp.float32),
    )(part, dgr, b)


def kernel(x, edge_index, W1, b1, W2, b2, W3, b3):
    f32 = jnp.float32
    x_p = jnp.zeros((NP, D_IN), f32).at[:N].set(x)
    pad = EP - E
    pad_ar = jnp.arange(pad, dtype=jnp.int32)
    src4 = jnp.concatenate([edge_index[0], pad_ar % N]).reshape(NW, 2, HCH, CHUNK)
    dst4 = jnp.concatenate([edge_index[1], N + (pad_ar % 16)]).reshape(NW, 2, HCH, CHUNK)
    srcA, srcB = src4[:, 0], src4[:, 1]
    dstA, dstB = dst4[:, 0], dst4[:, 1]
    z128 = jnp.zeros((NP, D_HID), f32)
    z64 = jnp.zeros((NP, D_OUT), f32)

    degp = _deg_call(dstA, dstB)
    dgr = (degp[0] + degp[1] + 1.0)[:, None]

    hs1 = _tc_first(x_p, W1, dgr)
    part1 = _agg128(hs1, srcA, srcB, dstA, dstB, z128)
    hs2 = _tc_mid(part1, dgr, b1.reshape(1, -1), W2)
    part2 = _agg128(hs2, srcA, srcB, dstA, dstB, z128)
    hs3 = _tc_mid(part2, dgr, b2.reshape(1, -1), W3)
    part3 = _agg64(hs3, srcA, srcB, dstA, dstB, z64)
    return _tc_fin(part3, dgr, b3.reshape(1, -1))
